# Initial kernel scaffold; baseline (speedup 1.0000x reference)
import jax
import jax.numpy as jnp
from jax.experimental import pallas as pl


def kernel(features_0, features_1, edge_index_o0, edge_index_o1, edge_index_i0, edge_index_i1, simlar, fc_w0, fc_b0, fc_w1, fc_b1, gat_w, a_src, a_dst, sa_w1, sa_b1, sa_w2, proj_w, proj_b):
    raise NotImplementedError("write your pallas kernel here")


# speedup vs baseline: 27.6423x; 27.6423x over previous
import jax
import jax.numpy as jnp
from jax import lax
from jax.experimental import pallas as pl
from jax.experimental.pallas import tpu as pltpu
from jax.experimental.pallas import tpu_sc as plsc

N = 10000
E = 320000
D_IN = 128
HID = 64
SA_HID = 128
OUT = 64

NC = 2
NS = 16
NW = NC * NS
EW = E // NW
C = 80
NCH = EW // C
RWA = 632
RWT = N - 15 * RWA
NBUF2 = 6
BLK = 1000
NB = N // BLK

_f32 = jnp.float32


def _split_copy(s, copy_fn):
    off = pl.multiple_of(s * RWA, 8)

    @pl.when(s < 15)
    def _():
        copy_fn(off, RWA)

    @pl.when(s == 15)
    def _():
        copy_fn(15 * RWA, RWT)


def _tc_prologue_body(f0, w0, b0, gw, a2, sim, t1_o, wh_o, h0_o, s_o):
    h0 = jnp.maximum(jnp.dot(f0[...], w0[...],
                             preferred_element_type=_f32) + b0[...], 0.0)
    wh = jnp.dot(h0, gw[...], preferred_element_type=_f32)
    t1_o[...] = sim[...] * h0
    wh_o[...] = wh
    h0_o[...] = h0
    s_o[...] = jnp.dot(wh, a2[...], preferred_element_type=_f32)


def _tc_prologue(f0, w0, b0, gw, a2, sim):
    full = lambda s: pl.BlockSpec(s, lambda i: tuple(0 for _ in s))
    return pl.pallas_call(
        _tc_prologue_body,
        grid=(NB,),
        in_specs=[
            pl.BlockSpec((BLK, D_IN), lambda i: (i, 0)),
            full((D_IN, HID)), full((1, HID)), full((HID, HID)),
            full((HID, 2)),
            pl.BlockSpec((BLK, 1), lambda i: (i, 0)),
        ],
        out_specs=[
            pl.BlockSpec((BLK, HID), lambda i: (i, 0)),
            pl.BlockSpec((BLK, HID), lambda i: (i, 0)),
            pl.BlockSpec((BLK, HID), lambda i: (i, 0)),
            pl.BlockSpec((BLK, 2), lambda i: (i, 0)),
        ],
        out_shape=[
            jax.ShapeDtypeStruct((N, HID), _f32),
            jax.ShapeDtypeStruct((N, HID), _f32),
            jax.ShapeDtypeStruct((N, HID), _f32),
            jax.ShapeDtypeStruct((N, 2), _f32),
        ],
    )(f0, w0, b0, gw, a2, sim)


def _sc1_body(t1_h, wh_h, srcs_h, dsts_h, sim_h, ssrc_h, sdst_h, z2_h,
              prP_h, gatP_h, degP_h, denP_h,
              src_v, dst_v, sim_v, ssrc_v, sdst_v, ex_v, deg_l, den_l,
              gbuf, acc, gsem, ssem):
    c = lax.axis_index("c")
    s = lax.axis_index("s")
    w = c * NS + s
    zv = jnp.zeros((16,), _f32)

    pltpu.sync_copy(sim_h, sim_v)
    pltpu.sync_copy(ssrc_h, ssrc_v)
    pltpu.sync_copy(sdst_h, sdst_v)

    def pipe(tab_h, scalars, scale):
        pltpu.async_copy(tab_h.at[src_v.at[0]], gbuf.at[0], gsem.at[0])

        def _iter(j, _):
            b = lax.rem(j, 2)
            nb = 1 - b
            m = j + 1

            @pl.when(m < NCH)
            def _():
                @pl.when(j >= 1)
                def _():
                    pltpu.make_async_copy(gbuf.at[nb],
                                          acc.at[dst_v.at[j - 1]],
                                          ssem.at[nb]).wait()
                pltpu.async_copy(tab_h.at[src_v.at[m]], gbuf.at[nb],
                                 gsem.at[nb])
            if scalars is not None:
                scalars(j)
            pltpu.make_async_copy(tab_h.at[src_v.at[j]], gbuf.at[b],
                                  gsem.at[b]).wait()
            if scale is not None:
                scale(j, b)
            pltpu.async_copy(gbuf.at[b], acc.at[dst_v.at[j]], ssem.at[b],
                             add=True)
            return 0
        lax.fori_loop(0, NCH, _iter, 0)

        for j in (NCH - 2, NCH - 1):
            pltpu.make_async_copy(gbuf.at[j % 2], acc.at[dst_v.at[j]],
                                  ssem.at[j % 2]).wait()

    def scalars(j):
        for k in range(C // 16):
            src16 = src_v[j, pl.ds(k * 16, 16)]
            dst16 = dst_v[j, pl.ds(k * 16, 16)]
            w16 = plsc.load_gather(sim_v, [src16])
            ta = plsc.load_gather(ssrc_v, [src16])
            tb = plsc.load_gather(sdst_v, [dst16])
            t = ta + tb
            lr = jnp.where(t >= 0.0, t, t * 0.2)
            ex = jnp.exp(lr + w16)
            ex_v[j, pl.ds(k * 16, 16)] = ex
            dhi = lax.shift_right_logical(dst16, 4)
            dlo = lax.bitwise_and(dst16, 15)
            plsc.addupdate_scatter(deg_l, [dhi, dlo], w16)
            plsc.addupdate_scatter(den_l, [dhi, dlo], ex)

    def scale(j, b):
        j16 = jnp.broadcast_to(j, (16,))

        def _se(e, _):
            exb = plsc.load_gather(ex_v, [j16, jnp.broadcast_to(e, (16,))])
            for q in range(4):
                gbuf[b, e, pl.ds(q * 16, 16)] = (
                    gbuf[b, e, pl.ds(q * 16, 16)] * exb)
            return 0
        lax.fori_loop(0, C, _se, 0)

    for g in range(2):
        _split_copy(s, lambda off, ln: pltpu.sync_copy(
            z2_h.at[pl.ds(0, ln)], acc.at[pl.ds(off, ln)]))

        def _zero(t, _):
            deg_l[t] = zv
            den_l[t] = zv
            return 0
        lax.fori_loop(0, N // 16, _zero, 0)

        pltpu.sync_copy(srcs_h.at[g, w], src_v)
        pltpu.sync_copy(dsts_h.at[g, w], dst_v)
        plsc.subcore_barrier()

        pipe(t1_h, scalars, None)
        plsc.subcore_barrier()
        _split_copy(s, lambda off, ln: pltpu.sync_copy(
            acc.at[pl.ds(off, ln)], prP_h.at[g, c, pl.ds(off, ln)]))
        pltpu.sync_copy(deg_l, degP_h.at[g, c, s])
        pltpu.sync_copy(den_l, denP_h.at[g, c, s])
        plsc.subcore_barrier()

        _split_copy(s, lambda off, ln: pltpu.sync_copy(
            z2_h.at[pl.ds(0, ln)], acc.at[pl.ds(off, ln)]))
        plsc.subcore_barrier()
        pipe(wh_h, None, scale)
        plsc.subcore_barrier()
        _split_copy(s, lambda off, ln: pltpu.sync_copy(
            acc.at[pl.ds(off, ln)], gatP_h.at[g, c, pl.ds(off, ln)]))
        plsc.subcore_barrier()


def _sc_pass1(t1, wh, srcs, dsts, sim, ssrc, sdst, z2):
    i32 = jnp.int32
    return pl.kernel(
        _sc1_body,
        out_type=[
            jax.ShapeDtypeStruct((2, NC, N, HID), _f32),
            jax.ShapeDtypeStruct((2, NC, N, HID), _f32),
            jax.ShapeDtypeStruct((2, NC, NS, N // 16, 16), _f32),
            jax.ShapeDtypeStruct((2, NC, NS, N // 16, 16), _f32),
        ],
        mesh=plsc.VectorSubcoreMesh(core_axis_name="c", subcore_axis_name="s"),
        compiler_params=pltpu.CompilerParams(needs_layout_passes=False,
                                             use_tc_tiling_on_sc=False),
        scratch_types=[
            pltpu.VMEM((NCH, C), i32),
            pltpu.VMEM((NCH, C), i32),
            pltpu.VMEM((N,), _f32),
            pltpu.VMEM((N,), _f32),
            pltpu.VMEM((N,), _f32),
            pltpu.VMEM((NCH, C), _f32),
            pltpu.VMEM((N // 16, 16), _f32),
            pltpu.VMEM((N // 16, 16), _f32),
            pltpu.VMEM((2, C, HID), _f32),
            pltpu.VMEM_SHARED((N, HID), _f32),
            pltpu.SemaphoreType.DMA((2,)),
            pltpu.SemaphoreType.DMA((2,)),
        ],
    )(t1, wh, srcs, dsts, sim, ssrc, sdst, z2)


def _tc_combine_body(prP, gatP, degT, denT, h0, sim,
                     y0_o, y1_o, gat_o, degc_o):
    h0b = h0[...]
    simb = sim[...]
    for g in range(2):
        ag = prP[g, 0] + prP[g, 1]
        deg = jnp.sum(degT[:, 32 * g:32 * g + 32], axis=1,
                      keepdims=True) + 1e-6
        out1 = 0.15 * h0b + 0.85 * ag / deg
        y = simb * out1
        if g == 0:
            y0_o[...] = y
        else:
            y1_o[...] = y
        degc_o[:, g:g + 1] = deg
        den = jnp.sum(denT[:, 32 * g:32 * g + 32], axis=1,
                      keepdims=True) + 1e-9
        gg = gatP[g, 0] + gatP[g, 1]
        x = gg / den
        gat_o[g, ...] = jnp.where(x > 0.0, x,
                                  jnp.exp(jnp.minimum(x, 0.0)) - 1.0)


def _tc_combine(prP, gatP, degT, denT, h0, sim):
    return pl.pallas_call(
        _tc_combine_body,
        grid=(NB,),
        in_specs=[
            pl.BlockSpec((2, NC, BLK, HID), lambda i: (0, 0, i, 0)),
            pl.BlockSpec((2, NC, BLK, HID), lambda i: (0, 0, i, 0)),
            pl.BlockSpec((BLK, 64), lambda i: (i, 0)),
            pl.BlockSpec((BLK, 64), lambda i: (i, 0)),
            pl.BlockSpec((BLK, HID), lambda i: (i, 0)),
            pl.BlockSpec((BLK, 1), lambda i: (i, 0)),
        ],
        out_specs=[
            pl.BlockSpec((BLK, HID), lambda i: (i, 0)),
            pl.BlockSpec((BLK, HID), lambda i: (i, 0)),
            pl.BlockSpec((2, BLK, HID), lambda i: (0, i, 0)),
            pl.BlockSpec((BLK, 2), lambda i: (i, 0)),
        ],
        out_shape=[
            jax.ShapeDtypeStruct((N, HID), _f32),
            jax.ShapeDtypeStruct((N, HID), _f32),
            jax.ShapeDtypeStruct((2, N, HID), _f32),
            jax.ShapeDtypeStruct((N, 2), _f32),
        ],
    )(prP, gatP, degT, denT, h0, sim)


def _sc2_body(y0_h, y1_h, srcs_h, dsts_h, z2_h,
              agg2P_h, src_v, dst_v, gbuf, acc, gsem, ssem):
    c = lax.axis_index("c")
    s = lax.axis_index("s")
    w = c * NS + s

    for g in range(2):
        y_h = (y0_h, y1_h)[g]
        _split_copy(s, lambda off, ln: pltpu.sync_copy(
            z2_h.at[pl.ds(0, ln)], acc.at[pl.ds(off, ln)]))
        pltpu.sync_copy(srcs_h.at[g, w], src_v)
        pltpu.sync_copy(dsts_h.at[g, w], dst_v)
        plsc.subcore_barrier()

        for m in range(3):
            pltpu.async_copy(y_h.at[src_v.at[m]], gbuf.at[m], gsem.at[m])

        def _iter(j, _):
            b = lax.rem(j, NBUF2)
            pltpu.make_async_copy(y_h.at[src_v.at[j]],
                                  gbuf.at[b], gsem.at[b]).wait()
            pltpu.async_copy(gbuf.at[b], acc.at[dst_v.at[j]], ssem.at[b],
                             add=True)
            m = j + 3

            @pl.when(m < NCH)
            def _():
                bm = lax.rem(m, NBUF2)

                @pl.when(j >= 3)
                def _():
                    pltpu.make_async_copy(gbuf.at[bm],
                                          acc.at[dst_v.at[j - 3]],
                                          ssem.at[bm]).wait()
                pltpu.async_copy(y_h.at[src_v.at[m]],
                                 gbuf.at[bm], gsem.at[bm])
            return 0
        lax.fori_loop(0, NCH, _iter, 0)

        for j in range(NCH - 6, NCH):
            pltpu.make_async_copy(gbuf.at[j % NBUF2], acc.at[dst_v.at[j]],
                                  ssem.at[j % NBUF2]).wait()
        plsc.subcore_barrier()

        _split_copy(s, lambda off, ln: pltpu.sync_copy(
            acc.at[pl.ds(off, ln)], agg2P_h.at[g, c, pl.ds(off, ln)]))
        plsc.subcore_barrier()


def _sc_pass2(y0, y1, srcs, dsts, z2):
    i32 = jnp.int32
    return pl.kernel(
        _sc2_body,
        out_type=jax.ShapeDtypeStruct((2, NC, N, HID), _f32),
        mesh=plsc.VectorSubcoreMesh(core_axis_name="c", subcore_axis_name="s"),
        compiler_params=pltpu.CompilerParams(needs_layout_passes=False,
                                             use_tc_tiling_on_sc=False),
        scratch_types=[
            pltpu.VMEM((NCH, C), i32),
            pltpu.VMEM((NCH, C), i32),
            pltpu.VMEM((NBUF2, C, HID), _f32),
            pltpu.VMEM_SHARED((N, HID), _f32),
            pltpu.SemaphoreType.DMA((NBUF2,)),
            pltpu.SemaphoreType.DMA((NBUF2,)),
        ],
    )(y0, y1, srcs, dsts, z2)


def _tc_epilogue_body(agg2P, degc, h0, gat, w1, b1, w2, pw, pb,
                      a_o, h_o, sm):
    p = pl.program_id(0)

    h0b = h0[...]
    gat0 = gat[0]
    gat1 = gat[1]
    agg = agg2P[...]

    def feat(g):
        ag = agg[g, 0] + agg[g, 1]
        return 0.15 * h0b + 0.85 * ag / degc[:, g:g + 1]

    def usum(z):
        t = jnp.tanh(jnp.dot(z, w1[...], preferred_element_type=_f32) + b1[...])
        return jnp.sum(jnp.dot(t, w2[...], preferred_element_type=_f32))

    def beta(k):
        t0 = sm[2 * k] * (1.0 / N)
        t1 = sm[2 * k + 1] * (1.0 / N)
        m = jnp.maximum(t0, t1)
        e0 = jnp.exp(t0 - m)
        e1 = jnp.exp(t1 - m)
        return e0 / (e0 + e1), e1 / (e0 + e1)

    @pl.when(p == 0)
    def _():
        @pl.when(pl.program_id(1) == 0)
        def _():
            for k in range(6):
                sm[k] = 0.0
        f0 = feat(0)
        f1 = feat(1)
        sm[0] += usum(f0)
        sm[1] += usum(f1)
        sm[2] += usum(gat0)
        sm[3] += usum(gat1)
        a_o[...] = f0
        h_o[...] = f1

    @pl.when(p == 1)
    def _():
        bo0, bo1 = beta(0)
        bi0, bi1 = beta(1)
        ho = bo0 * feat(0) + bo1 * feat(1)
        hi = bi0 * gat0 + bi1 * gat1
        sm[4] += usum(ho)
        sm[5] += usum(hi)
        a_o[...] = ho
        h_o[...] = hi

    @pl.when(p == 2)
    def _():
        bo0, bo1 = beta(0)
        bi0, bi1 = beta(1)
        bh0, bh1 = beta(2)
        ho = bo0 * feat(0) + bo1 * feat(1)
        hi = bi0 * gat0 + bi1 * gat1
        h = bh0 * ho + bh1 * hi
        h_o[...] = h
        a_o[...] = jnp.dot(h, pw[...], preferred_element_type=_f32) + pb[...]


def _tc_epilogue(agg2P, degc, h0, gat, w1, b1, w2, pw, pb):
    full = lambda s: pl.BlockSpec(s, lambda p, i: tuple(0 for _ in s))
    return pl.pallas_call(
        _tc_epilogue_body,
        grid=(3, NB),
        in_specs=[
            pl.BlockSpec((2, NC, BLK, HID), lambda p, i: (0, 0, i, 0)),
            pl.BlockSpec((BLK, 2), lambda p, i: (i, 0)),
            pl.BlockSpec((BLK, HID), lambda p, i: (i, 0)),
            pl.BlockSpec((2, BLK, HID), lambda p, i: (0, i, 0)),
            full((HID, SA_HID)), full((1, SA_HID)), full((SA_HID, 1)),
            full((HID, OUT)), full((1, OUT)),
        ],
        out_specs=[
            pl.BlockSpec((BLK, OUT), lambda p, i: (i, 0)),
            pl.BlockSpec((BLK, HID), lambda p, i: (i, 0)),
        ],
        out_shape=[
            jax.ShapeDtypeStruct((N, OUT), _f32),
            jax.ShapeDtypeStruct((N, HID), _f32),
        ],
        scratch_shapes=[pltpu.SMEM((8,), _f32)],
    )(agg2P, degc, h0, gat, w1, b1, w2, pw, pb)


def kernel(features_0, features_1, edge_index_o0, edge_index_o1,
           edge_index_i0, edge_index_i1, simlar,
           fc_w0, fc_b0, fc_w1, fc_b1, gat_w, a_src, a_dst,
           sa_w1, sa_b1, sa_w2, proj_w, proj_b):
    sim2 = simlar.reshape(N, 1)
    a2 = jnp.stack([a_src, a_dst], axis=1)
    t1, wh, h0, s_tab = _tc_prologue(features_0, fc_w0, fc_b0.reshape(1, HID),
                                     gat_w, a2, sim2)
    ssrc = s_tab[:, 0]
    sdst = s_tab[:, 1]

    srcs = jnp.stack([edge_index_o0[0], edge_index_o1[0]]
                     ).astype(jnp.int32).reshape(2, NW, NCH, C)
    dsts = jnp.stack([edge_index_o0[1], edge_index_o1[1]]
                     ).astype(jnp.int32).reshape(2, NW, NCH, C)
    z2 = jnp.zeros((RWA, HID), _f32)

    prP, gatP, degP, denP = _sc_pass1(t1, wh, srcs, dsts, simlar, ssrc, sdst,
                                      z2)
    degT = degP.reshape(2, NW, N).transpose(2, 0, 1).reshape(N, 2 * NW)
    denT = denP.reshape(2, NW, N).transpose(2, 0, 1).reshape(N, 2 * NW)

    y0, y1, gat, degc = _tc_combine(prP, gatP, degT, denT, h0, sim2)

    agg2P = _sc_pass2(y0, y1, srcs, dsts, z2)

    a, h = _tc_epilogue(agg2P, degc, h0, gat,
                        sa_w1, sa_b1.reshape(1, SA_HID), sa_w2,
                        proj_w, proj_b.reshape(1, OUT))
    return (a, h)


# speedup vs baseline: 31.4673x; 1.1384x over previous
import jax
import jax.numpy as jnp
from jax import lax
from jax.experimental import pallas as pl
from jax.experimental.pallas import tpu as pltpu
from jax.experimental.pallas import tpu_sc as plsc

N = 10000
E = 320000
D_IN = 128
HID = 64
SA_HID = 128
OUT = 64

NC = 2
NS = 16
NW = NC * NS
EW = E // NW
C = 80
NCH = EW // C
RWA = 632
RWT = N - 15 * RWA
NBUF2 = 6
BLK = 1000
NB = N // BLK

_f32 = jnp.float32


def _split_copy(s, copy_fn):
    off = pl.multiple_of(s * RWA, 8)

    @pl.when(s < 15)
    def _():
        copy_fn(off, RWA)

    @pl.when(s == 15)
    def _():
        copy_fn(15 * RWA, RWT)


def _tc_prologue_body(f0, w0, b0, gw, a2, sim, t1_o, wh_o, h0_o, s_o):
    h0 = jnp.maximum(jnp.dot(f0[...], w0[...],
                             preferred_element_type=_f32) + b0[...], 0.0)
    wh = jnp.dot(h0, gw[...], preferred_element_type=_f32)
    t1_o[...] = sim[...] * h0
    wh_o[...] = wh
    h0_o[...] = h0
    s_o[...] = jnp.dot(wh, a2[...], preferred_element_type=_f32)


def _tc_prologue(f0, w0, b0, gw, a2, sim):
    full = lambda s: pl.BlockSpec(s, lambda i: tuple(0 for _ in s))
    return pl.pallas_call(
        _tc_prologue_body,
        grid=(NB,),
        in_specs=[
            pl.BlockSpec((BLK, D_IN), lambda i: (i, 0)),
            full((D_IN, HID)), full((1, HID)), full((HID, HID)),
            full((HID, 2)),
            pl.BlockSpec((BLK, 1), lambda i: (i, 0)),
        ],
        out_specs=[
            pl.BlockSpec((BLK, HID), lambda i: (i, 0)),
            pl.BlockSpec((BLK, HID), lambda i: (i, 0)),
            pl.BlockSpec((BLK, HID), lambda i: (i, 0)),
            pl.BlockSpec((BLK, 2), lambda i: (i, 0)),
        ],
        out_shape=[
            jax.ShapeDtypeStruct((N, HID), _f32),
            jax.ShapeDtypeStruct((N, HID), _f32),
            jax.ShapeDtypeStruct((N, HID), _f32),
            jax.ShapeDtypeStruct((N, 2), _f32),
        ],
    )(f0, w0, b0, gw, a2, sim)


def _sc1_body(t1_h, wh_h, srcs_h, dsts_h, sim_h, ssrc_h, sdst_h, z2_h,
              prP_h, gatP_h, degP_h, denP_h,
              src_v, dst_v, sim_v, ssrc_v, sdst_v, ex_v, deg_l, den_l,
              gbuf, acc, gsem, ssem):
    c = lax.axis_index("c")
    s = lax.axis_index("s")
    w = c * NS + s
    zv = jnp.zeros((16,), _f32)

    pltpu.sync_copy(sim_h, sim_v)
    pltpu.sync_copy(ssrc_h, ssrc_v)
    pltpu.sync_copy(sdst_h, sdst_v)

    def pipe(tab_h, prework, scale):
        for m in range(3):
            pltpu.async_copy(tab_h.at[src_v.at[m]], gbuf.at[m], gsem.at[m])

        def _iter(j, _):
            b = lax.rem(j, 4)
            if prework is not None:
                prework(j)
            pltpu.make_async_copy(tab_h.at[src_v.at[j]], gbuf.at[b],
                                  gsem.at[b]).wait()
            if scale is not None:
                scale(j, b)
            pltpu.async_copy(gbuf.at[b], acc.at[dst_v.at[j]], ssem.at[b],
                             add=True)
            m = j + 3

            @pl.when(m < NCH)
            def _():
                bm = lax.rem(m, 4)

                @pl.when(j >= 1)
                def _():
                    pltpu.make_async_copy(gbuf.at[bm],
                                          acc.at[dst_v.at[j - 1]],
                                          ssem.at[bm]).wait()
                pltpu.async_copy(tab_h.at[src_v.at[m]], gbuf.at[bm],
                                 gsem.at[bm])
            return 0
        lax.fori_loop(0, NCH, _iter, 0)

        for j in range(NCH - 4, NCH):
            pltpu.make_async_copy(gbuf.at[j % 4], acc.at[dst_v.at[j]],
                                  ssem.at[j % 4]).wait()

    def edge_exp(j, k):
        src16 = src_v[j, pl.ds(k * 16, 16)]
        dst16 = dst_v[j, pl.ds(k * 16, 16)]
        w16 = plsc.load_gather(sim_v, [src16])
        ta = plsc.load_gather(ssrc_v, [src16])
        tb = plsc.load_gather(sdst_v, [dst16])
        t = ta + tb
        lr = jnp.where(t >= 0.0, t, t * 0.2)
        return dst16, w16, jnp.exp(lr + w16)

    def scalars(j):
        for k in range(C // 16):
            dst16, w16, ex = edge_exp(j, k)
            dhi = lax.shift_right_logical(dst16, 4)
            dlo = lax.bitwise_and(dst16, 15)
            plsc.addupdate_scatter(deg_l, [dhi, dlo], w16)
            plsc.addupdate_scatter(den_l, [dhi, dlo], ex)

    def scalars_b(j):
        for k in range(C // 16):
            _, _, ex = edge_exp(j, k)
            ex_v[pl.ds(k * 16, 16)] = ex

    def scale(j, b):
        def _se(e4, _):
            e0 = e4 * 4
            for u in range(4):
                exb = plsc.load_gather(
                    ex_v, [jnp.broadcast_to(e0 + u, (16,))])
                for q in range(4):
                    gbuf[b, e0 + u, pl.ds(q * 16, 16)] = (
                        gbuf[b, e0 + u, pl.ds(q * 16, 16)] * exb)
            return 0
        lax.fori_loop(0, C // 4, _se, 0)

    for g in range(2):
        _split_copy(s, lambda off, ln: pltpu.sync_copy(
            z2_h.at[pl.ds(0, ln)], acc.at[pl.ds(off, ln)]))

        def _zero(t, _):
            deg_l[t] = zv
            den_l[t] = zv
            return 0
        lax.fori_loop(0, N // 16, _zero, 0)

        pltpu.sync_copy(srcs_h.at[g, w], src_v)
        pltpu.sync_copy(dsts_h.at[g, w], dst_v)
        plsc.subcore_barrier()

        pipe(t1_h, scalars, None)
        plsc.subcore_barrier()
        _split_copy(s, lambda off, ln: pltpu.sync_copy(
            acc.at[pl.ds(off, ln)], prP_h.at[g, c, pl.ds(off, ln)]))
        pltpu.sync_copy(deg_l, degP_h.at[g, c, s])
        pltpu.sync_copy(den_l, denP_h.at[g, c, s])
        plsc.subcore_barrier()

        _split_copy(s, lambda off, ln: pltpu.sync_copy(
            z2_h.at[pl.ds(0, ln)], acc.at[pl.ds(off, ln)]))
        plsc.subcore_barrier()
        pipe(wh_h, scalars_b, scale)
        plsc.subcore_barrier()
        _split_copy(s, lambda off, ln: pltpu.sync_copy(
            acc.at[pl.ds(off, ln)], gatP_h.at[g, c, pl.ds(off, ln)]))
        plsc.subcore_barrier()


def _sc_pass1(t1, wh, srcs, dsts, sim, ssrc, sdst, z2):
    i32 = jnp.int32
    return pl.kernel(
        _sc1_body,
        out_type=[
            jax.ShapeDtypeStruct((2, NC, N, HID), _f32),
            jax.ShapeDtypeStruct((2, NC, N, HID), _f32),
            jax.ShapeDtypeStruct((2, NC, NS, N // 16, 16), _f32),
            jax.ShapeDtypeStruct((2, NC, NS, N // 16, 16), _f32),
        ],
        mesh=plsc.VectorSubcoreMesh(core_axis_name="c", subcore_axis_name="s"),
        compiler_params=pltpu.CompilerParams(needs_layout_passes=False,
                                             use_tc_tiling_on_sc=False),
        scratch_types=[
            pltpu.VMEM((NCH, C), i32),
            pltpu.VMEM((NCH, C), i32),
            pltpu.VMEM((N,), _f32),
            pltpu.VMEM((N,), _f32),
            pltpu.VMEM((N,), _f32),
            pltpu.VMEM((C,), _f32),
            pltpu.VMEM((N // 16, 16), _f32),
            pltpu.VMEM((N // 16, 16), _f32),
            pltpu.VMEM((4, C, HID), _f32),
            pltpu.VMEM_SHARED((N, HID), _f32),
            pltpu.SemaphoreType.DMA((4,)),
            pltpu.SemaphoreType.DMA((4,)),
        ],
    )(t1, wh, srcs, dsts, sim, ssrc, sdst, z2)


def _tc_combine_body(prP, gatP, degT, denT, h0, sim,
                     y0_o, y1_o, gat_o, degc_o):
    h0b = h0[...]
    simb = sim[...]
    for g in range(2):
        ag = prP[g, 0] + prP[g, 1]
        deg = jnp.sum(degT[:, 32 * g:32 * g + 32], axis=1,
                      keepdims=True) + 1e-6
        out1 = 0.15 * h0b + 0.85 * ag / deg
        y = simb * out1
        if g == 0:
            y0_o[...] = y
        else:
            y1_o[...] = y
        degc_o[:, g:g + 1] = deg
        den = jnp.sum(denT[:, 32 * g:32 * g + 32], axis=1,
                      keepdims=True) + 1e-9
        gg = gatP[g, 0] + gatP[g, 1]
        x = gg / den
        gat_o[g, ...] = jnp.where(x > 0.0, x,
                                  jnp.exp(jnp.minimum(x, 0.0)) - 1.0)


def _tc_combine(prP, gatP, degT, denT, h0, sim):
    return pl.pallas_call(
        _tc_combine_body,
        grid=(NB,),
        in_specs=[
            pl.BlockSpec((2, NC, BLK, HID), lambda i: (0, 0, i, 0)),
            pl.BlockSpec((2, NC, BLK, HID), lambda i: (0, 0, i, 0)),
            pl.BlockSpec((BLK, 64), lambda i: (i, 0)),
            pl.BlockSpec((BLK, 64), lambda i: (i, 0)),
            pl.BlockSpec((BLK, HID), lambda i: (i, 0)),
            pl.BlockSpec((BLK, 1), lambda i: (i, 0)),
        ],
        out_specs=[
            pl.BlockSpec((BLK, HID), lambda i: (i, 0)),
            pl.BlockSpec((BLK, HID), lambda i: (i, 0)),
            pl.BlockSpec((2, BLK, HID), lambda i: (0, i, 0)),
            pl.BlockSpec((BLK, 2), lambda i: (i, 0)),
        ],
        out_shape=[
            jax.ShapeDtypeStruct((N, HID), _f32),
            jax.ShapeDtypeStruct((N, HID), _f32),
            jax.ShapeDtypeStruct((2, N, HID), _f32),
            jax.ShapeDtypeStruct((N, 2), _f32),
        ],
    )(prP, gatP, degT, denT, h0, sim)


def _sc2_body(y0_h, y1_h, srcs_h, dsts_h, z2_h,
              agg2P_h, src_v, dst_v, gbuf, acc, gsem, ssem):
    c = lax.axis_index("c")
    s = lax.axis_index("s")
    w = c * NS + s

    for g in range(2):
        y_h = (y0_h, y1_h)[g]
        _split_copy(s, lambda off, ln: pltpu.sync_copy(
            z2_h.at[pl.ds(0, ln)], acc.at[pl.ds(off, ln)]))
        pltpu.sync_copy(srcs_h.at[g, w], src_v)
        pltpu.sync_copy(dsts_h.at[g, w], dst_v)
        plsc.subcore_barrier()

        for m in range(3):
            pltpu.async_copy(y_h.at[src_v.at[m]], gbuf.at[m], gsem.at[m])

        def _iter(j, _):
            b = lax.rem(j, NBUF2)
            pltpu.make_async_copy(y_h.at[src_v.at[j]],
                                  gbuf.at[b], gsem.at[b]).wait()
            pltpu.async_copy(gbuf.at[b], acc.at[dst_v.at[j]], ssem.at[b],
                             add=True)
            m = j + 3

            @pl.when(m < NCH)
            def _():
                bm = lax.rem(m, NBUF2)

                @pl.when(j >= 3)
                def _():
                    pltpu.make_async_copy(gbuf.at[bm],
                                          acc.at[dst_v.at[j - 3]],
                                          ssem.at[bm]).wait()
                pltpu.async_copy(y_h.at[src_v.at[m]],
                                 gbuf.at[bm], gsem.at[bm])
            return 0
        lax.fori_loop(0, NCH, _iter, 0)

        for j in range(NCH - 6, NCH):
            pltpu.make_async_copy(gbuf.at[j % NBUF2], acc.at[dst_v.at[j]],
                                  ssem.at[j % NBUF2]).wait()
        plsc.subcore_barrier()

        _split_copy(s, lambda off, ln: pltpu.sync_copy(
            acc.at[pl.ds(off, ln)], agg2P_h.at[g, c, pl.ds(off, ln)]))
        plsc.subcore_barrier()


def _sc_pass2(y0, y1, srcs, dsts, z2):
    i32 = jnp.int32
    return pl.kernel(
        _sc2_body,
        out_type=jax.ShapeDtypeStruct((2, NC, N, HID), _f32),
        mesh=plsc.VectorSubcoreMesh(core_axis_name="c", subcore_axis_name="s"),
        compiler_params=pltpu.CompilerParams(needs_layout_passes=False,
                                             use_tc_tiling_on_sc=False),
        scratch_types=[
            pltpu.VMEM((NCH, C), i32),
            pltpu.VMEM((NCH, C), i32),
            pltpu.VMEM((NBUF2, C, HID), _f32),
            pltpu.VMEM_SHARED((N, HID), _f32),
            pltpu.SemaphoreType.DMA((NBUF2,)),
            pltpu.SemaphoreType.DMA((NBUF2,)),
        ],
    )(y0, y1, srcs, dsts, z2)


def _tc_epilogue_body(agg2P, degc, h0, gat, w1, b1, w2, pw, pb,
                      a_o, h_o, sm):
    p = pl.program_id(0)

    h0b = h0[...]
    gat0 = gat[0]
    gat1 = gat[1]
    agg = agg2P[...]

    def feat(g):
        ag = agg[g, 0] + agg[g, 1]
        return 0.15 * h0b + 0.85 * ag / degc[:, g:g + 1]

    def usum(z):
        t = jnp.tanh(jnp.dot(z, w1[...], preferred_element_type=_f32) + b1[...])
        return jnp.sum(jnp.dot(t, w2[...], preferred_element_type=_f32))

    def beta(k):
        t0 = sm[2 * k] * (1.0 / N)
        t1 = sm[2 * k + 1] * (1.0 / N)
        m = jnp.maximum(t0, t1)
        e0 = jnp.exp(t0 - m)
        e1 = jnp.exp(t1 - m)
        return e0 / (e0 + e1), e1 / (e0 + e1)

    @pl.when(p == 0)
    def _():
        @pl.when(pl.program_id(1) == 0)
        def _():
            for k in range(6):
                sm[k] = 0.0
        f0 = feat(0)
        f1 = feat(1)
        sm[0] += usum(f0)
        sm[1] += usum(f1)
        sm[2] += usum(gat0)
        sm[3] += usum(gat1)
        a_o[...] = f0
        h_o[...] = f1

    @pl.when(p == 1)
    def _():
        bo0, bo1 = beta(0)
        bi0, bi1 = beta(1)
        ho = bo0 * feat(0) + bo1 * feat(1)
        hi = bi0 * gat0 + bi1 * gat1
        sm[4] += usum(ho)
        sm[5] += usum(hi)
        a_o[...] = ho
        h_o[...] = hi

    @pl.when(p == 2)
    def _():
        bo0, bo1 = beta(0)
        bi0, bi1 = beta(1)
        bh0, bh1 = beta(2)
        ho = bo0 * feat(0) + bo1 * feat(1)
        hi = bi0 * gat0 + bi1 * gat1
        h = bh0 * ho + bh1 * hi
        h_o[...] = h
        a_o[...] = jnp.dot(h, pw[...], preferred_element_type=_f32) + pb[...]


def _tc_epilogue(agg2P, degc, h0, gat, w1, b1, w2, pw, pb):
    full = lambda s: pl.BlockSpec(s, lambda p, i: tuple(0 for _ in s))
    return pl.pallas_call(
        _tc_epilogue_body,
        grid=(3, NB),
        in_specs=[
            pl.BlockSpec((2, NC, BLK, HID), lambda p, i: (0, 0, i, 0)),
            pl.BlockSpec((BLK, 2), lambda p, i: (i, 0)),
            pl.BlockSpec((BLK, HID), lambda p, i: (i, 0)),
            pl.BlockSpec((2, BLK, HID), lambda p, i: (0, i, 0)),
            full((HID, SA_HID)), full((1, SA_HID)), full((SA_HID, 1)),
            full((HID, OUT)), full((1, OUT)),
        ],
        out_specs=[
            pl.BlockSpec((BLK, OUT), lambda p, i: (i, 0)),
            pl.BlockSpec((BLK, HID), lambda p, i: (i, 0)),
        ],
        out_shape=[
            jax.ShapeDtypeStruct((N, OUT), _f32),
            jax.ShapeDtypeStruct((N, HID), _f32),
        ],
        scratch_shapes=[pltpu.SMEM((8,), _f32)],
    )(agg2P, degc, h0, gat, w1, b1, w2, pw, pb)


def kernel(features_0, features_1, edge_index_o0, edge_index_o1,
           edge_index_i0, edge_index_i1, simlar,
           fc_w0, fc_b0, fc_w1, fc_b1, gat_w, a_src, a_dst,
           sa_w1, sa_b1, sa_w2, proj_w, proj_b):
    sim2 = simlar.reshape(N, 1)
    a2 = jnp.stack([a_src, a_dst], axis=1)
    t1, wh, h0, s_tab = _tc_prologue(features_0, fc_w0, fc_b0.reshape(1, HID),
                                     gat_w, a2, sim2)
    ssrc = s_tab[:, 0]
    sdst = s_tab[:, 1]

    srcs = jnp.stack([edge_index_o0[0], edge_index_o1[0]]
                     ).astype(jnp.int32).reshape(2, NW, NCH, C)
    dsts = jnp.stack([edge_index_o0[1], edge_index_o1[1]]
                     ).astype(jnp.int32).reshape(2, NW, NCH, C)
    z2 = jnp.zeros((RWA, HID), _f32)

    prP, gatP, degP, denP = _sc_pass1(t1, wh, srcs, dsts, simlar, ssrc, sdst,
                                      z2)
    degT = degP.reshape(2, NW, N).transpose(2, 0, 1).reshape(N, 2 * NW)
    denT = denP.reshape(2, NW, N).transpose(2, 0, 1).reshape(N, 2 * NW)

    y0, y1, gat, degc = _tc_combine(prP, gatP, degT, denT, h0, sim2)

    agg2P = _sc_pass2(y0, y1, srcs, dsts, z2)

    a, h = _tc_epilogue(agg2P, degc, h0, gat,
                        sa_w1, sa_b1.reshape(1, SA_HID), sa_w2,
                        proj_w, proj_b.reshape(1, OUT))
    return (a, h)


# speedup vs baseline: 33.6420x; 1.0691x over previous
import jax
import jax.numpy as jnp
from jax import lax
from jax.experimental import pallas as pl
from jax.experimental.pallas import tpu as pltpu
from jax.experimental.pallas import tpu_sc as plsc

N = 10000
E = 320000
D_IN = 128
HID = 64
SA_HID = 128
OUT = 64

NC = 2
NS = 16
EW = E // NS
C = 80
NCH = EW // C
RWA = 632
RWT = N - 15 * RWA
BLK = 1000
NB = N // BLK

_f32 = jnp.float32


def _split_copy(s, copy_fn):
    off = pl.multiple_of(s * RWA, 8)

    @pl.when(s < 15)
    def _():
        copy_fn(off, RWA)

    @pl.when(s == 15)
    def _():
        copy_fn(15 * RWA, RWT)


def _edge_pipe(c, s, eidx_h, eidx, tab_h, gbuf, acc, gsem, ssem, isem,
               prework, scale):
    def idx_fetch(m):
        bi = lax.rem(m, 8)
        pltpu.async_copy(eidx_h.at[c, s, m], eidx.at[bi], isem.at[bi])

    def idx_wait(m):
        bi = lax.rem(m, 8)
        pltpu.make_async_copy(eidx_h.at[c, s, m], eidx.at[bi],
                              isem.at[bi]).wait()

    for m in range(6):
        idx_fetch(m)
    for m in range(3):
        idx_wait(m)
        pltpu.async_copy(tab_h.at[eidx.at[m, 0]], gbuf.at[m], gsem.at[m])

    def _iter(j, _):
        b = lax.rem(j, 4)
        ji = lax.rem(j, 8)
        if prework is not None:
            prework(ji)
        pltpu.make_async_copy(tab_h.at[eidx.at[ji, 0]], gbuf.at[b],
                              gsem.at[b]).wait()
        if scale is not None:
            scale(b)
        pltpu.async_copy(gbuf.at[b], acc.at[eidx.at[ji, 1]], ssem.at[b],
                         add=True)

        m6 = j + 6

        @pl.when(m6 < NCH)
        def _():
            idx_fetch(m6)

        m3 = j + 3

        @pl.when(m3 < NCH)
        def _():
            bm = lax.rem(m3, 4)

            @pl.when(j >= 1)
            def _():
                pltpu.make_async_copy(gbuf.at[bm],
                                      acc.at[eidx.at[lax.rem(j - 1, 8), 1]],
                                      ssem.at[bm]).wait()
            idx_wait(m3)
            pltpu.async_copy(tab_h.at[eidx.at[lax.rem(m3, 8), 0]],
                             gbuf.at[bm], gsem.at[bm])
        return 0
    lax.fori_loop(0, NCH, _iter, 0)

    for j in range(NCH - 4, NCH):
        pltpu.make_async_copy(gbuf.at[j % 4], acc.at[eidx.at[j % 8, 1]],
                              ssem.at[j % 4]).wait()


def _tc_prologue_body(f0, w0, b0, gw, a2, sim, t1_o, wh_o, h0_o, s_o):
    h0 = jnp.maximum(jnp.dot(f0[...], w0[...],
                             preferred_element_type=_f32) + b0[...], 0.0)
    wh = jnp.dot(h0, gw[...], preferred_element_type=_f32)
    t1_o[...] = sim[...] * h0
    wh_o[...] = wh
    h0_o[...] = h0
    s_o[...] = jnp.dot(wh, a2[...], preferred_element_type=_f32)


def _tc_prologue(f0, w0, b0, gw, a2, sim):
    full = lambda s: pl.BlockSpec(s, lambda i: tuple(0 for _ in s))
    return pl.pallas_call(
        _tc_prologue_body,
        grid=(NB,),
        in_specs=[
            pl.BlockSpec((BLK, D_IN), lambda i: (i, 0)),
            full((D_IN, HID)), full((1, HID)), full((HID, HID)),
            full((HID, 2)),
            pl.BlockSpec((BLK, 1), lambda i: (i, 0)),
        ],
        out_specs=[
            pl.BlockSpec((BLK, HID), lambda i: (i, 0)),
            pl.BlockSpec((BLK, HID), lambda i: (i, 0)),
            pl.BlockSpec((BLK, HID), lambda i: (i, 0)),
            pl.BlockSpec((BLK, 2), lambda i: (i, 0)),
        ],
        out_shape=[
            jax.ShapeDtypeStruct((N, HID), _f32),
            jax.ShapeDtypeStruct((N, HID), _f32),
            jax.ShapeDtypeStruct((N, HID), _f32),
            jax.ShapeDtypeStruct((N, 2), _f32),
        ],
    )(f0, w0, b0, gw, a2, sim)


def _sc1_body(t1_h, wh_h, eidx_h, sim_h, ssrc_h, sdst_h, z2_h, zdeg_h, iota_h,
              prA_h, gatn_h, degD_h, denD_h,
              eidx, sim_v, ssrc_v, sdst_v, ex_v, deg_l, den_l, iota_v,
              gbuf, acc, deg_sp, den_sp, gsem, ssem, isem):
    c = lax.axis_index("c")
    s = lax.axis_index("s")
    zv = jnp.zeros((16,), _f32)

    pltpu.sync_copy(sim_h, sim_v)
    pltpu.sync_copy(ssrc_h, ssrc_v)
    pltpu.sync_copy(sdst_h, sdst_v)
    pltpu.sync_copy(iota_h, iota_v)

    @pl.when(s == 0)
    def _():
        pltpu.sync_copy(zdeg_h, deg_sp)
        pltpu.sync_copy(zdeg_h, den_sp)

    def _zero(t, _):
        deg_l[t] = zv
        den_l[t] = zv
        return 0
    lax.fori_loop(0, N // 16, _zero, 0)

    _split_copy(s, lambda off, ln: pltpu.sync_copy(
        z2_h.at[pl.ds(0, ln)], acc.at[pl.ds(off, ln)]))
    plsc.subcore_barrier()

    def edge_exp(ji, k):
        src16 = eidx[ji, 0, pl.ds(k * 16, 16)]
        dst16 = eidx[ji, 1, pl.ds(k * 16, 16)]
        w16 = plsc.load_gather(sim_v, [src16])
        ta = plsc.load_gather(ssrc_v, [src16])
        tb = plsc.load_gather(sdst_v, [dst16])
        t = ta + tb
        lr = jnp.where(t >= 0.0, t, t * 0.2)
        return dst16, w16, jnp.exp(lr + w16)

    def scalars(ji):
        for k in range(C // 16):
            dst16, w16, ex = edge_exp(ji, k)
            dhi = lax.shift_right_logical(dst16, 4)
            dlo = lax.bitwise_and(dst16, 15)
            plsc.addupdate_scatter(deg_l, [dhi, dlo], w16)
            plsc.addupdate_scatter(den_l, [dhi, dlo], ex)

    def scalars_b(ji):
        for k in range(C // 16):
            _, _, ex = edge_exp(ji, k)
            ex_v[pl.ds(k * 16, 16)] = ex

    def scale(b):
        def _se(e4, _):
            e0 = e4 * 4
            for u in range(4):
                exb = plsc.load_gather(
                    ex_v, [jnp.broadcast_to(e0 + u, (16,))])
                for q in range(4):
                    gbuf[b, e0 + u, pl.ds(q * 16, 16)] = (
                        gbuf[b, e0 + u, pl.ds(q * 16, 16)] * exb)
            return 0
        lax.fori_loop(0, C // 4, _se, 0)

    _edge_pipe(c, s, eidx_h, eidx, t1_h, gbuf, acc, gsem, ssem, isem,
               scalars, None)
    pltpu.sync_copy(deg_l, deg_sp.at[iota_v], add=True)
    pltpu.sync_copy(den_l, den_sp.at[iota_v], add=True)
    plsc.subcore_barrier()
    _split_copy(s, lambda off, ln: pltpu.sync_copy(
        acc.at[pl.ds(off, ln)], prA_h.at[c, pl.ds(off, ln)]))

    @pl.when(s == 0)
    def _():
        pltpu.sync_copy(deg_sp, degD_h.at[c])
        pltpu.sync_copy(den_sp, denD_h.at[c])
    plsc.subcore_barrier()
    _split_copy(s, lambda off, ln: pltpu.sync_copy(
        z2_h.at[pl.ds(0, ln)], acc.at[pl.ds(off, ln)]))
    plsc.subcore_barrier()

    _edge_pipe(c, s, eidx_h, eidx, wh_h, gbuf, acc, gsem, ssem, isem,
               scalars_b, scale)
    plsc.subcore_barrier()
    _split_copy(s, lambda off, ln: pltpu.sync_copy(
        acc.at[pl.ds(off, ln)], gatn_h.at[c, pl.ds(off, ln)]))


def _sc_pass1(t1, wh, eidx, sim, ssrc, sdst, z2, zdeg, iota):
    i32 = jnp.int32
    return pl.kernel(
        _sc1_body,
        out_type=[
            jax.ShapeDtypeStruct((NC, N, HID), _f32),
            jax.ShapeDtypeStruct((NC, N, HID), _f32),
            jax.ShapeDtypeStruct((NC, N // 16, 16), _f32),
            jax.ShapeDtypeStruct((NC, N // 16, 16), _f32),
        ],
        mesh=plsc.VectorSubcoreMesh(core_axis_name="c", subcore_axis_name="s"),
        compiler_params=pltpu.CompilerParams(needs_layout_passes=False,
                                             use_tc_tiling_on_sc=False),
        scratch_types=[
            pltpu.VMEM((8, 2, C), i32),
            pltpu.VMEM((N,), _f32),
            pltpu.VMEM((N,), _f32),
            pltpu.VMEM((N,), _f32),
            pltpu.VMEM((C,), _f32),
            pltpu.VMEM((N // 16, 16), _f32),
            pltpu.VMEM((N // 16, 16), _f32),
            pltpu.VMEM((N // 16,), i32),
            pltpu.VMEM((4, C, HID), _f32),
            pltpu.VMEM_SHARED((N, HID), _f32),
            pltpu.VMEM_SHARED((N // 16, 16), _f32),
            pltpu.VMEM_SHARED((N // 16, 16), _f32),
            pltpu.SemaphoreType.DMA((4,)),
            pltpu.SemaphoreType.DMA((4,)),
            pltpu.SemaphoreType.DMA((8,)),
        ],
    )(t1, wh, eidx, sim, ssrc, sdst, z2, zdeg, iota)


def _tc_combine_body(prA, gatn, degc, denc, h0, sim, y0_o, y1_o, gat_o):
    h0b = h0[...]
    simb = sim[...]
    for g in range(2):
        deg = degc[:, g:g + 1] + 1e-6
        out1 = 0.15 * h0b + 0.85 * prA[g] / deg
        y = simb * out1
        if g == 0:
            y0_o[...] = y
        else:
            y1_o[...] = y
        den = denc[:, g:g + 1] + 1e-9
        x = gatn[g] / den
        gat_o[g, ...] = jnp.where(x > 0.0, x,
                                  jnp.exp(jnp.minimum(x, 0.0)) - 1.0)


def _tc_combine(prA, gatn, degc, denc, h0, sim):
    return pl.pallas_call(
        _tc_combine_body,
        grid=(NB,),
        in_specs=[
            pl.BlockSpec((2, BLK, HID), lambda i: (0, i, 0)),
            pl.BlockSpec((2, BLK, HID), lambda i: (0, i, 0)),
            pl.BlockSpec((BLK, 2), lambda i: (i, 0)),
            pl.BlockSpec((BLK, 2), lambda i: (i, 0)),
            pl.BlockSpec((BLK, HID), lambda i: (i, 0)),
            pl.BlockSpec((BLK, 1), lambda i: (i, 0)),
        ],
        out_specs=[
            pl.BlockSpec((BLK, HID), lambda i: (i, 0)),
            pl.BlockSpec((BLK, HID), lambda i: (i, 0)),
            pl.BlockSpec((2, BLK, HID), lambda i: (0, i, 0)),
        ],
        out_shape=[
            jax.ShapeDtypeStruct((N, HID), _f32),
            jax.ShapeDtypeStruct((N, HID), _f32),
            jax.ShapeDtypeStruct((2, N, HID), _f32),
        ],
    )(prA, gatn, degc, denc, h0, sim)


def _sc2_body(y0_h, y1_h, eidx_h, z2_h, agg2_h,
              eidx, gbuf, acc, gsem, ssem, isem):
    c = lax.axis_index("c")
    s = lax.axis_index("s")

    _split_copy(s, lambda off, ln: pltpu.sync_copy(
        z2_h.at[pl.ds(0, ln)], acc.at[pl.ds(off, ln)]))
    plsc.subcore_barrier()

    @pl.when(c == 0)
    def _():
        _edge_pipe(c, s, eidx_h, eidx, y0_h, gbuf, acc, gsem, ssem, isem,
                   None, None)

    @pl.when(c == 1)
    def _():
        _edge_pipe(c, s, eidx_h, eidx, y1_h, gbuf, acc, gsem, ssem, isem,
                   None, None)
    plsc.subcore_barrier()
    _split_copy(s, lambda off, ln: pltpu.sync_copy(
        acc.at[pl.ds(off, ln)], agg2_h.at[c, pl.ds(off, ln)]))


def _sc_pass2(y0, y1, eidx, z2):
    i32 = jnp.int32
    return pl.kernel(
        _sc2_body,
        out_type=jax.ShapeDtypeStruct((NC, N, HID), _f32),
        mesh=plsc.VectorSubcoreMesh(core_axis_name="c", subcore_axis_name="s"),
        compiler_params=pltpu.CompilerParams(needs_layout_passes=False,
                                             use_tc_tiling_on_sc=False),
        scratch_types=[
            pltpu.VMEM((8, 2, C), i32),
            pltpu.VMEM((4, C, HID), _f32),
            pltpu.VMEM_SHARED((N, HID), _f32),
            pltpu.SemaphoreType.DMA((4,)),
            pltpu.SemaphoreType.DMA((4,)),
            pltpu.SemaphoreType.DMA((8,)),
        ],
    )(y0, y1, eidx, z2)


def _tc_epilogue_body(agg2, degc, h0, gat, w1, b1, w2, pw, pb, a_o, h_o, sm):
    p = pl.program_id(0)

    h0b = h0[...]
    gat0 = gat[0]
    gat1 = gat[1]

    def feat(g):
        deg = degc[:, g:g + 1] + 1e-6
        return 0.15 * h0b + 0.85 * agg2[g] / deg

    def usum(z):
        t = jnp.tanh(jnp.dot(z, w1[...], preferred_element_type=_f32) + b1[...])
        return jnp.sum(jnp.dot(t, w2[...], preferred_element_type=_f32))

    def beta(k):
        t0 = sm[2 * k] * (1.0 / N)
        t1 = sm[2 * k + 1] * (1.0 / N)
        m = jnp.maximum(t0, t1)
        e0 = jnp.exp(t0 - m)
        e1 = jnp.exp(t1 - m)
        return e0 / (e0 + e1), e1 / (e0 + e1)

    @pl.when(p == 0)
    def _():
        @pl.when(pl.program_id(1) == 0)
        def _():
            for k in range(6):
                sm[k] = 0.0
        f0 = feat(0)
        f1 = feat(1)
        sm[0] += usum(f0)
        sm[1] += usum(f1)
        sm[2] += usum(gat0)
        sm[3] += usum(gat1)
        a_o[...] = f0
        h_o[...] = f1

    @pl.when(p == 1)
    def _():
        bo0, bo1 = beta(0)
        bi0, bi1 = beta(1)
        ho = bo0 * feat(0) + bo1 * feat(1)
        hi = bi0 * gat0 + bi1 * gat1
        sm[4] += usum(ho)
        sm[5] += usum(hi)
        a_o[...] = ho
        h_o[...] = hi

    @pl.when(p == 2)
    def _():
        bo0, bo1 = beta(0)
        bi0, bi1 = beta(1)
        bh0, bh1 = beta(2)
        ho = bo0 * feat(0) + bo1 * feat(1)
        hi = bi0 * gat0 + bi1 * gat1
        h = bh0 * ho + bh1 * hi
        h_o[...] = h
        a_o[...] = jnp.dot(h, pw[...], preferred_element_type=_f32) + pb[...]


def _tc_epilogue(agg2, degc, h0, gat, w1, b1, w2, pw, pb):
    full = lambda s: pl.BlockSpec(s, lambda p, i: tuple(0 for _ in s))
    return pl.pallas_call(
        _tc_epilogue_body,
        grid=(3, NB),
        in_specs=[
            pl.BlockSpec((2, BLK, HID), lambda p, i: (0, i, 0)),
            pl.BlockSpec((BLK, 2), lambda p, i: (i, 0)),
            pl.BlockSpec((BLK, HID), lambda p, i: (i, 0)),
            pl.BlockSpec((2, BLK, HID), lambda p, i: (0, i, 0)),
            full((HID, SA_HID)), full((1, SA_HID)), full((SA_HID, 1)),
            full((HID, OUT)), full((1, OUT)),
        ],
        out_specs=[
            pl.BlockSpec((BLK, OUT), lambda p, i: (i, 0)),
            pl.BlockSpec((BLK, HID), lambda p, i: (i, 0)),
        ],
        out_shape=[
            jax.ShapeDtypeStruct((N, OUT), _f32),
            jax.ShapeDtypeStruct((N, HID), _f32),
        ],
        scratch_shapes=[pltpu.SMEM((8,), _f32)],
    )(agg2, degc, h0, gat, w1, b1, w2, pw, pb)


def kernel(features_0, features_1, edge_index_o0, edge_index_o1,
           edge_index_i0, edge_index_i1, simlar,
           fc_w0, fc_b0, fc_w1, fc_b1, gat_w, a_src, a_dst,
           sa_w1, sa_b1, sa_w2, proj_w, proj_b):
    sim2 = simlar.reshape(N, 1)
    a2 = jnp.stack([a_src, a_dst], axis=1)
    t1, wh, h0, s_tab = _tc_prologue(features_0, fc_w0, fc_b0.reshape(1, HID),
                                     gat_w, a2, sim2)
    ssrc = s_tab[:, 0]
    sdst = s_tab[:, 1]

    eidx = jnp.stack([edge_index_o0, edge_index_o1]
                     ).astype(jnp.int32).reshape(2, 2, NS, NCH, C
                                                 ).transpose(0, 2, 3, 1, 4)
    z2 = jnp.zeros((RWA, HID), _f32)
    zdeg = jnp.zeros((N // 16, 16), _f32)
    iota = jnp.arange(N // 16, dtype=jnp.int32)

    prA, gatn, degD, denD = _sc_pass1(t1, wh, eidx, simlar, ssrc, sdst,
                                      z2, zdeg, iota)
    degc = degD.reshape(2, N).T
    denc = denD.reshape(2, N).T

    y0, y1, gat = _tc_combine(prA, gatn, degc, denc, h0, sim2)

    agg2 = _sc_pass2(y0, y1, eidx, z2)

    a, h = _tc_epilogue(agg2, degc, h0, gat,
                        sa_w1, sa_b1.reshape(1, SA_HID), sa_w2,
                        proj_w, proj_b.reshape(1, OUT))
    return (a, h)


# speedup vs baseline: 35.1804x; 1.0457x over previous
import jax
import jax.numpy as jnp
from jax import lax
from jax.experimental import pallas as pl
from jax.experimental.pallas import tpu as pltpu
from jax.experimental.pallas import tpu_sc as plsc

N = 10000
E = 320000
D_IN = 128
HID = 64
SA_HID = 128
OUT = 64

NC = 2
NS = 16
EW = E // NS
C = 80
NCH = EW // C
RWA = 632
RWT = N - 15 * RWA
BLK = 2000
NB = N // BLK

_f32 = jnp.float32


def _split_copy(s, copy_fn):
    off = pl.multiple_of(s * RWA, 8)

    @pl.when(s < 15)
    def _():
        copy_fn(off, RWA)

    @pl.when(s == 15)
    def _():
        copy_fn(15 * RWA, RWT)


def _edge_pipe(c, s, eidx_h, eidx, tab_h, gbuf, acc, gsem, ssem, isem,
               prework, scale):
    def idx_fetch(m):
        bi = lax.rem(m, 8)
        pltpu.async_copy(eidx_h.at[c, s, m], eidx.at[bi], isem.at[bi])

    def idx_wait(m):
        bi = lax.rem(m, 8)
        pltpu.make_async_copy(eidx_h.at[c, s, m], eidx.at[bi],
                              isem.at[bi]).wait()

    for m in range(6):
        idx_fetch(m)
    for m in range(3):
        idx_wait(m)
        pltpu.async_copy(tab_h.at[eidx.at[m, 0]], gbuf.at[m], gsem.at[m])

    def _iter(j, _):
        b = lax.rem(j, 4)
        ji = lax.rem(j, 8)
        if prework is not None:
            prework(ji)
        pltpu.make_async_copy(tab_h.at[eidx.at[ji, 0]], gbuf.at[b],
                              gsem.at[b]).wait()
        if scale is not None:
            scale(b)
        pltpu.async_copy(gbuf.at[b], acc.at[eidx.at[ji, 1]], ssem.at[b],
                         add=True)

        m6 = j + 6

        @pl.when(m6 < NCH)
        def _():
            idx_fetch(m6)

        m3 = j + 3

        @pl.when(m3 < NCH)
        def _():
            bm = lax.rem(m3, 4)

            @pl.when(j >= 1)
            def _():
                pltpu.make_async_copy(gbuf.at[bm],
                                      acc.at[eidx.at[lax.rem(j - 1, 8), 1]],
                                      ssem.at[bm]).wait()
            idx_wait(m3)
            pltpu.async_copy(tab_h.at[eidx.at[lax.rem(m3, 8), 0]],
                             gbuf.at[bm], gsem.at[bm])
        return 0
    lax.fori_loop(0, NCH, _iter, 0)

    for j in range(NCH - 4, NCH):
        pltpu.make_async_copy(gbuf.at[j % 4], acc.at[eidx.at[j % 8, 1]],
                              ssem.at[j % 4]).wait()


def _tc_prologue_body(f0, w0, b0, gw, a2, sim, t1_o, wh_o, h0_o, s_o):
    h0 = jnp.maximum(jnp.dot(f0[...], w0[...],
                             preferred_element_type=_f32) + b0[...], 0.0)
    wh = jnp.dot(h0, gw[...], preferred_element_type=_f32)
    t1_o[...] = sim[...] * h0
    wh_o[...] = wh
    h0_o[...] = h0
    s_o[...] = jnp.dot(wh, a2[...], preferred_element_type=_f32)


def _tc_prologue(f0, w0, b0, gw, a2, sim):
    full = lambda s: pl.BlockSpec(s, lambda i: tuple(0 for _ in s))
    return pl.pallas_call(
        _tc_prologue_body,
        grid=(NB,),
        in_specs=[
            pl.BlockSpec((BLK, D_IN), lambda i: (i, 0)),
            full((D_IN, HID)), full((1, HID)), full((HID, HID)),
            full((HID, 2)),
            pl.BlockSpec((BLK, 1), lambda i: (i, 0)),
        ],
        out_specs=[
            pl.BlockSpec((BLK, HID), lambda i: (i, 0)),
            pl.BlockSpec((BLK, HID), lambda i: (i, 0)),
            pl.BlockSpec((BLK, HID), lambda i: (i, 0)),
            pl.BlockSpec((BLK, 2), lambda i: (i, 0)),
        ],
        out_shape=[
            jax.ShapeDtypeStruct((N, HID), _f32),
            jax.ShapeDtypeStruct((N, HID), _f32),
            jax.ShapeDtypeStruct((N, HID), _f32),
            jax.ShapeDtypeStruct((N, 2), _f32),
        ],
    )(f0, w0, b0, gw, a2, sim)


def _sc1_body(t1_h, wh_h, eidx_h, sim_h, ssrc_h, sdst_h, z2_h, zdeg_h, iota_h,
              prA_h, gatn_h, degD_h, denD_h,
              eidx, sim_v, ssrc_v, sdst_v, ex_v, deg_l, den_l, iota_v,
              gbuf, acc, deg_sp, den_sp, gsem, ssem, isem):
    c = lax.axis_index("c")
    s = lax.axis_index("s")
    zv = jnp.zeros((16,), _f32)

    pltpu.sync_copy(sim_h, sim_v)
    pltpu.sync_copy(ssrc_h, ssrc_v)
    pltpu.sync_copy(sdst_h, sdst_v)
    pltpu.sync_copy(iota_h, iota_v)

    @pl.when(s == 0)
    def _():
        pltpu.sync_copy(zdeg_h, deg_sp)
        pltpu.sync_copy(zdeg_h, den_sp)

    def _zero(t, _):
        deg_l[t] = zv
        den_l[t] = zv
        return 0
    lax.fori_loop(0, N // 16, _zero, 0)

    _split_copy(s, lambda off, ln: pltpu.sync_copy(
        z2_h.at[pl.ds(0, ln)], acc.at[pl.ds(off, ln)]))
    plsc.subcore_barrier()

    def edge_exp(ji, k):
        src16 = eidx[ji, 0, pl.ds(k * 16, 16)]
        dst16 = eidx[ji, 1, pl.ds(k * 16, 16)]
        w16 = plsc.load_gather(sim_v, [src16])
        ta = plsc.load_gather(ssrc_v, [src16])
        tb = plsc.load_gather(sdst_v, [dst16])
        t = ta + tb
        lr = jnp.where(t >= 0.0, t, t * 0.2)
        return dst16, w16, jnp.exp(lr + w16)

    def scalars(ji):
        for k in range(C // 16):
            dst16, w16, ex = edge_exp(ji, k)
            dhi = lax.shift_right_logical(dst16, 4)
            dlo = lax.bitwise_and(dst16, 15)
            plsc.addupdate_scatter(deg_l, [dhi, dlo], w16)
            plsc.addupdate_scatter(den_l, [dhi, dlo], ex)

    def scalars_b(ji):
        for k in range(C // 16):
            _, _, ex = edge_exp(ji, k)
            ex_v[pl.ds(k * 16, 16)] = ex

    def scale(b):
        def _se(e4, _):
            e0 = e4 * 4
            for u in range(4):
                exb = plsc.load_gather(
                    ex_v, [jnp.broadcast_to(e0 + u, (16,))])
                for q in range(4):
                    gbuf[b, e0 + u, pl.ds(q * 16, 16)] = (
                        gbuf[b, e0 + u, pl.ds(q * 16, 16)] * exb)
            return 0
        lax.fori_loop(0, C // 4, _se, 0)

    _edge_pipe(c, s, eidx_h, eidx, t1_h, gbuf, acc, gsem, ssem, isem,
               scalars, None)
    pltpu.sync_copy(deg_l, deg_sp.at[iota_v], add=True)
    pltpu.sync_copy(den_l, den_sp.at[iota_v], add=True)
    plsc.subcore_barrier()
    _split_copy(s, lambda off, ln: pltpu.sync_copy(
        acc.at[pl.ds(off, ln)], prA_h.at[c, pl.ds(off, ln)]))

    @pl.when(s == 0)
    def _():
        pltpu.sync_copy(deg_sp, degD_h.at[c])
        pltpu.sync_copy(den_sp, denD_h.at[c])
    plsc.subcore_barrier()
    _split_copy(s, lambda off, ln: pltpu.sync_copy(
        z2_h.at[pl.ds(0, ln)], acc.at[pl.ds(off, ln)]))
    plsc.subcore_barrier()

    _edge_pipe(c, s, eidx_h, eidx, wh_h, gbuf, acc, gsem, ssem, isem,
               scalars_b, scale)
    plsc.subcore_barrier()
    _split_copy(s, lambda off, ln: pltpu.sync_copy(
        acc.at[pl.ds(off, ln)], gatn_h.at[c, pl.ds(off, ln)]))


def _sc_pass1(t1, wh, eidx, sim, ssrc, sdst, z2, zdeg, iota):
    i32 = jnp.int32
    return pl.kernel(
        _sc1_body,
        out_type=[
            jax.ShapeDtypeStruct((NC, N, HID), _f32),
            jax.ShapeDtypeStruct((NC, N, HID), _f32),
            jax.ShapeDtypeStruct((NC, N // 16, 16), _f32),
            jax.ShapeDtypeStruct((NC, N // 16, 16), _f32),
        ],
        mesh=plsc.VectorSubcoreMesh(core_axis_name="c", subcore_axis_name="s"),
        compiler_params=pltpu.CompilerParams(needs_layout_passes=False,
                                             use_tc_tiling_on_sc=False),
        scratch_types=[
            pltpu.VMEM((8, 2, C), i32),
            pltpu.VMEM((N,), _f32),
            pltpu.VMEM((N,), _f32),
            pltpu.VMEM((N,), _f32),
            pltpu.VMEM((C,), _f32),
            pltpu.VMEM((N // 16, 16), _f32),
            pltpu.VMEM((N // 16, 16), _f32),
            pltpu.VMEM((N // 16,), i32),
            pltpu.VMEM((4, C, HID), _f32),
            pltpu.VMEM_SHARED((N, HID), _f32),
            pltpu.VMEM_SHARED((N // 16, 16), _f32),
            pltpu.VMEM_SHARED((N // 16, 16), _f32),
            pltpu.SemaphoreType.DMA((4,)),
            pltpu.SemaphoreType.DMA((4,)),
            pltpu.SemaphoreType.DMA((8,)),
        ],
    )(t1, wh, eidx, sim, ssrc, sdst, z2, zdeg, iota)


def _tc_combine_body(prA, degc, h0, sim, y0_o, y1_o):
    h0b = h0[...]
    simb = sim[...]
    for g in range(2):
        deg = degc[:, g:g + 1] + 1e-6
        out1 = 0.15 * h0b + 0.85 * prA[g] / deg
        y = simb * out1
        if g == 0:
            y0_o[...] = y
        else:
            y1_o[...] = y


def _tc_combine(prA, degc, h0, sim):
    return pl.pallas_call(
        _tc_combine_body,
        grid=(NB,),
        in_specs=[
            pl.BlockSpec((2, BLK, HID), lambda i: (0, i, 0)),
            pl.BlockSpec((BLK, 2), lambda i: (i, 0)),
            pl.BlockSpec((BLK, HID), lambda i: (i, 0)),
            pl.BlockSpec((BLK, 1), lambda i: (i, 0)),
        ],
        out_specs=[
            pl.BlockSpec((BLK, HID), lambda i: (i, 0)),
            pl.BlockSpec((BLK, HID), lambda i: (i, 0)),
        ],
        out_shape=[
            jax.ShapeDtypeStruct((N, HID), _f32),
            jax.ShapeDtypeStruct((N, HID), _f32),
        ],
    )(prA, degc, h0, sim)


def _sc2_body(y0_h, y1_h, eidx_h, z2_h, agg2_h,
              eidx, gbuf, acc, gsem, ssem, isem):
    c = lax.axis_index("c")
    s = lax.axis_index("s")

    _split_copy(s, lambda off, ln: pltpu.sync_copy(
        z2_h.at[pl.ds(0, ln)], acc.at[pl.ds(off, ln)]))
    plsc.subcore_barrier()

    @pl.when(c == 0)
    def _():
        _edge_pipe(c, s, eidx_h, eidx, y0_h, gbuf, acc, gsem, ssem, isem,
                   None, None)

    @pl.when(c == 1)
    def _():
        _edge_pipe(c, s, eidx_h, eidx, y1_h, gbuf, acc, gsem, ssem, isem,
                   None, None)
    plsc.subcore_barrier()
    _split_copy(s, lambda off, ln: pltpu.sync_copy(
        acc.at[pl.ds(off, ln)], agg2_h.at[c, pl.ds(off, ln)]))


def _sc_pass2(y0, y1, eidx, z2):
    i32 = jnp.int32
    return pl.kernel(
        _sc2_body,
        out_type=jax.ShapeDtypeStruct((NC, N, HID), _f32),
        mesh=plsc.VectorSubcoreMesh(core_axis_name="c", subcore_axis_name="s"),
        compiler_params=pltpu.CompilerParams(needs_layout_passes=False,
                                             use_tc_tiling_on_sc=False),
        scratch_types=[
            pltpu.VMEM((8, 2, C), i32),
            pltpu.VMEM((4, C, HID), _f32),
            pltpu.VMEM_SHARED((N, HID), _f32),
            pltpu.SemaphoreType.DMA((4,)),
            pltpu.SemaphoreType.DMA((4,)),
            pltpu.SemaphoreType.DMA((8,)),
        ],
    )(y0, y1, eidx, z2)


def _tc_epilogue_body(agg2, degc, h0, gatn, denc, w1, b1, w2, pw, pb,
                      a_o, h_o, sm):
    p = pl.program_id(0)

    h0b = h0[...]

    def elu_g(g):
        x = gatn[g] / (denc[:, g:g + 1] + 1e-9)
        return jnp.where(x > 0.0, x, jnp.exp(jnp.minimum(x, 0.0)) - 1.0)

    gat0 = elu_g(0)
    gat1 = elu_g(1)

    def feat(g):
        deg = degc[:, g:g + 1] + 1e-6
        return 0.15 * h0b + 0.85 * agg2[g] / deg

    def usum(z):
        t = jnp.tanh(jnp.dot(z, w1[...], preferred_element_type=_f32) + b1[...])
        return jnp.sum(jnp.dot(t, w2[...], preferred_element_type=_f32))

    def beta(k):
        t0 = sm[2 * k] * (1.0 / N)
        t1 = sm[2 * k + 1] * (1.0 / N)
        m = jnp.maximum(t0, t1)
        e0 = jnp.exp(t0 - m)
        e1 = jnp.exp(t1 - m)
        return e0 / (e0 + e1), e1 / (e0 + e1)

    @pl.when(p == 0)
    def _():
        @pl.when(pl.program_id(1) == 0)
        def _():
            for k in range(6):
                sm[k] = 0.0
        f0 = feat(0)
        f1 = feat(1)
        sm[0] += usum(f0)
        sm[1] += usum(f1)
        sm[2] += usum(gat0)
        sm[3] += usum(gat1)
        a_o[...] = f0
        h_o[...] = f1

    @pl.when(p == 1)
    def _():
        bo0, bo1 = beta(0)
        bi0, bi1 = beta(1)
        ho = bo0 * feat(0) + bo1 * feat(1)
        hi = bi0 * gat0 + bi1 * gat1
        sm[4] += usum(ho)
        sm[5] += usum(hi)
        a_o[...] = ho
        h_o[...] = hi

    @pl.when(p == 2)
    def _():
        bo0, bo1 = beta(0)
        bi0, bi1 = beta(1)
        bh0, bh1 = beta(2)
        ho = bo0 * feat(0) + bo1 * feat(1)
        hi = bi0 * gat0 + bi1 * gat1
        h = bh0 * ho + bh1 * hi
        h_o[...] = h
        a_o[...] = jnp.dot(h, pw[...], preferred_element_type=_f32) + pb[...]


def _tc_epilogue(agg2, degc, h0, gatn, denc, w1, b1, w2, pw, pb):
    full = lambda s: pl.BlockSpec(s, lambda p, i: tuple(0 for _ in s))
    return pl.pallas_call(
        _tc_epilogue_body,
        grid=(3, NB),
        in_specs=[
            pl.BlockSpec((2, BLK, HID), lambda p, i: (0, i, 0)),
            pl.BlockSpec((BLK, 2), lambda p, i: (i, 0)),
            pl.BlockSpec((BLK, HID), lambda p, i: (i, 0)),
            pl.BlockSpec((2, BLK, HID), lambda p, i: (0, i, 0)),
            pl.BlockSpec((BLK, 2), lambda p, i: (i, 0)),
            full((HID, SA_HID)), full((1, SA_HID)), full((SA_HID, 1)),
            full((HID, OUT)), full((1, OUT)),
        ],
        out_specs=[
            pl.BlockSpec((BLK, OUT), lambda p, i: (i, 0)),
            pl.BlockSpec((BLK, HID), lambda p, i: (i, 0)),
        ],
        out_shape=[
            jax.ShapeDtypeStruct((N, OUT), _f32),
            jax.ShapeDtypeStruct((N, HID), _f32),
        ],
        scratch_shapes=[pltpu.SMEM((8,), _f32)],
    )(agg2, degc, h0, gatn, denc, w1, b1, w2, pw, pb)


def kernel(features_0, features_1, edge_index_o0, edge_index_o1,
           edge_index_i0, edge_index_i1, simlar,
           fc_w0, fc_b0, fc_w1, fc_b1, gat_w, a_src, a_dst,
           sa_w1, sa_b1, sa_w2, proj_w, proj_b):
    sim2 = simlar.reshape(N, 1)
    a2 = jnp.stack([a_src, a_dst], axis=1)
    t1, wh, h0, s_tab = _tc_prologue(features_0, fc_w0, fc_b0.reshape(1, HID),
                                     gat_w, a2, sim2)
    ssrc = s_tab[:, 0]
    sdst = s_tab[:, 1]

    eidx = jnp.stack([edge_index_o0, edge_index_o1]
                     ).astype(jnp.int32).reshape(2, 2, NS, NCH, C
                                                 ).transpose(0, 2, 3, 1, 4)
    z2 = jnp.zeros((RWA, HID), _f32)
    zdeg = jnp.zeros((N // 16, 16), _f32)
    iota = jnp.arange(N // 16, dtype=jnp.int32)

    prA, gatn, degD, denD = _sc_pass1(t1, wh, eidx, simlar, ssrc, sdst,
                                      z2, zdeg, iota)
    degc = degD.reshape(2, N).T
    denc = denD.reshape(2, N).T

    y0, y1 = _tc_combine(prA, degc, h0, sim2)

    agg2 = _sc_pass2(y0, y1, eidx, z2)

    a, h = _tc_epilogue(agg2, degc, h0, gatn, denc,
                        sa_w1, sa_b1.reshape(1, SA_HID), sa_w2,
                        proj_w, proj_b.reshape(1, OUT))
    return (a, h)
